# Initial kernel scaffold; baseline (speedup 1.0000x reference)
import jax
import jax.numpy as jnp
from jax.experimental import pallas as pl


def kernel(x, edge_index, W_emb, b_emb, W0, b0, W1, b1, W2, b2):
    raise NotImplementedError("write your pallas kernel here")


# speedup vs baseline: 5.9709x; 5.9709x over previous
import functools

import jax
import jax.numpy as jnp
from jax import lax
from jax.experimental import pallas as pl
from jax.experimental.pallas import tpu as pltpu
from jax.experimental.pallas import tpu_sc as plsc

N = 10000
NPAD = 10240
E = 160000
EPAD = 163840
NT = 32
CHUNKS = 40
CK = 128
STRIPE = NPAD // 16

_MESH = plsc.VectorSubcoreMesh(core_axis_name="c", subcore_axis_name="s")


@functools.partial(
    pl.kernel,
    out_type=jax.ShapeDtypeStruct((2, 2, NPAD, 128), jnp.float32),
    mesh=_MESH,
    scratch_types=[
        pltpu.VMEM((CHUNKS, CK), jnp.int32),
        pltpu.VMEM((CK, 128), jnp.float32),
        pltpu.VMEM((CK, 128), jnp.float32),
        pltpu.VMEM_SHARED((NPAD, 128), jnp.float32),
    ],
)
def _deg_kernel(src_hbm, dst_hbm, out_hbm, idx_v, ones_v, zeros_v, degs):
    c = lax.axis_index("c")
    s = lax.axis_index("s")
    wid = c * 16 + s

    def _fill(i, _):
        for k in range(128 // 16):
            ones_v[i, pl.ds(k * 16, 16)] = jnp.ones((16,), jnp.float32)
            zeros_v[i, pl.ds(k * 16, 16)] = jnp.zeros((16,), jnp.float32)
        return 0

    lax.fori_loop(0, CK, _fill, 0)

    base = s * STRIPE
    for phase, e_hbm in ((0, src_hbm), (1, dst_hbm)):
        pltpu.sync_copy(e_hbm.at[wid], idx_v)
        for z in range(STRIPE // CK):
            pltpu.sync_copy(zeros_v, degs.at[pl.ds(base + z * CK, CK)])
        plsc.subcore_barrier()

        def _scat(j, _):
            pltpu.sync_copy(ones_v, degs.at[idx_v.at[j]], add=True)
            return 0

        lax.fori_loop(0, CHUNKS, _scat, 0)
        plsc.subcore_barrier()
        pltpu.sync_copy(degs.at[pl.ds(base, STRIPE)],
                        out_hbm.at[c, phase, pl.ds(base, STRIPE)])
        plsc.subcore_barrier()


def _make_agg(nb):
    @functools.partial(
        pl.kernel,
        out_type=jax.ShapeDtypeStruct((2, nb, NPAD, 128), jnp.float32),
        mesh=_MESH,
        scratch_types=[
            pltpu.VMEM((CHUNKS, CK), jnp.int32),
            pltpu.VMEM((CHUNKS, CK), jnp.int32),
            pltpu.VMEM((CK, 128), jnp.float32),
            pltpu.VMEM((CK, 128), jnp.float32),
            pltpu.VMEM_SHARED((NPAD, 128), jnp.float32),
            pltpu.SemaphoreType.DMA,
        ],
    )
    def _agg_kernel(t_hbm, src_hbm, dst_hbm, out_hbm, sidx, didx, rows_v, zeros_v, aggs, sem):
        c = lax.axis_index("c")
        s = lax.axis_index("s")
        wid = c * 16 + s
        pltpu.sync_copy(src_hbm.at[wid], sidx)
        pltpu.sync_copy(dst_hbm.at[wid], didx)

        def _zero(i, _):
            for k in range(128 // 16):
                zeros_v[i, pl.ds(k * 16, 16)] = jnp.zeros((16,), jnp.float32)
            return 0

        lax.fori_loop(0, CK, _zero, 0)

        base = s * STRIPE
        for cb in range(nb):
            t_cb = t_hbm.at[cb]
            for z in range(STRIPE // CK):
                pltpu.sync_copy(zeros_v, aggs.at[pl.ds(base + z * CK, CK)])
            plsc.subcore_barrier()

            def _edge(j, _):
                pltpu.async_copy(t_cb.at[sidx.at[j]], rows_v, sem).wait()
                pltpu.sync_copy(rows_v, aggs.at[didx.at[j]], add=True)
                return 0

            lax.fori_loop(0, CHUNKS, _edge, 0)
            plsc.subcore_barrier()
            pltpu.sync_copy(aggs.at[pl.ds(base, STRIPE)],
                            out_hbm.at[c, cb, pl.ds(base, STRIPE)])
            plsc.subcore_barrier()

    return _agg_kernel


_agg4 = _make_agg(4)
_agg1 = _make_agg(1)


def _norms(deg_ref, row0, rows):
    ridx = lax.broadcasted_iota(jnp.int32, (rows, 1), 0) + row0
    valid = ridx < N
    dego = deg_ref[0, 0] + deg_ref[1, 0]
    degi = deg_ref[0, 1] + deg_ref[1, 1]
    ns = jnp.where(valid, lax.rsqrt(jnp.maximum(dego, 1.0)), 0.0)
    nd = jnp.where(valid, lax.rsqrt(jnp.maximum(degi, 1.0)), 0.0)
    return ns, nd


_MMR = 2560


def _mm1_body(deg_ref, x_ref, we_ref, be_ref, w0_ref, out_ref):
    i = pl.program_id(0)
    ns, _ = _norms(deg_ref, i * _MMR, _MMR)
    h = jnp.dot(x_ref[...], we_ref[...], preferred_element_type=jnp.float32)
    h = (h + be_ref[...]) * ns
    w0 = w0_ref[...]
    for cb in range(4):
        out_ref[cb] = jnp.dot(h, w0[:, cb * 128:(cb + 1) * 128],
                              preferred_element_type=jnp.float32)


def _mm1(degp, xp, W_emb, b_emb, W0):
    return pl.pallas_call(
        _mm1_body,
        grid=(NPAD // _MMR,),
        in_specs=[
            pl.BlockSpec((2, 2, _MMR, 1), lambda i: (0, 0, i, 0)),
            pl.BlockSpec((_MMR, 256), lambda i: (i, 0)),
            pl.BlockSpec((256, 512), lambda i: (0, 0)),
            pl.BlockSpec((1, 512), lambda i: (0, 0)),
            pl.BlockSpec((512, 512), lambda i: (0, 0)),
        ],
        out_specs=pl.BlockSpec((4, _MMR, 128), lambda i: (0, i, 0)),
        out_shape=jax.ShapeDtypeStruct((4, NPAD, 128), jnp.float32),
    )(degp, xp, W_emb, b_emb, W0)


def _mm_mid_body(out_nb, deg_ref, a_ref, b_ref, w_ref, out_ref):
    i = pl.program_id(0)
    ns, nd = _norms(deg_ref, i * _MMR, _MMR)
    acc = jnp.zeros((_MMR, 512 if out_nb == 4 else 128), jnp.float32)
    for cb in range(4):
        u = (a_ref[0, cb] + a_ref[1, cb]) * nd + b_ref[cb]
        u = jnp.maximum(u, 0.0) * ns
        acc = acc + jnp.dot(u, w_ref[cb], preferred_element_type=jnp.float32)
    for cb in range(out_nb):
        out_ref[cb] = acc[:, cb * 128:(cb + 1) * 128]


def _mm_mid(degp, a, b, w, out_nb):
    return pl.pallas_call(
        functools.partial(_mm_mid_body, out_nb),
        grid=(NPAD // _MMR,),
        in_specs=[
            pl.BlockSpec((2, 2, _MMR, 1), lambda i: (0, 0, i, 0)),
            pl.BlockSpec((2, 4, _MMR, 128), lambda i: (0, 0, i, 0)),
            pl.BlockSpec((4, 1, 128), lambda i: (0, 0, 0)),
            pl.BlockSpec((4, 128, 128 * out_nb), lambda i: (0, 0, 0)),
        ],
        out_specs=pl.BlockSpec((out_nb, _MMR, 128), lambda i: (0, i, 0)),
        out_shape=jax.ShapeDtypeStruct((out_nb, NPAD, 128), jnp.float32),
    )(degp, a, b, w)


_FINR = 2000


def _fin_body(deg_ref, a_ref, b_ref, out_ref):
    i = pl.program_id(0)
    _, nd = _norms(deg_ref, i * _FINR, _FINR)
    out_ref[...] = (a_ref[0] + a_ref[1]) * nd + b_ref[...]


def _fin(degp, a2, b2):
    return pl.pallas_call(
        _fin_body,
        grid=(N // _FINR,),
        in_specs=[
            pl.BlockSpec((2, 2, _FINR, 1), lambda i: (0, 0, i, 0)),
            pl.BlockSpec((2, _FINR, 128), lambda i: (0, i, 0)),
            pl.BlockSpec((1, 128), lambda i: (0, 0)),
        ],
        out_specs=pl.BlockSpec((_FINR, 128), lambda i: (i, 0)),
        out_shape=jax.ShapeDtypeStruct((N, 128), jnp.float32),
    )(degp, a2, b2)


def kernel(x, edge_index, W_emb, b_emb, W0, b0, W1, b1, W2, b2):
    src = edge_index[0].astype(jnp.int32)
    dst = edge_index[1].astype(jnp.int32)
    pad = N + (jnp.arange(EPAD - E, dtype=jnp.int32) % 16)
    srcp = jnp.concatenate([src, pad]).reshape(NT, CHUNKS, CK)
    dstp = jnp.concatenate([dst, pad]).reshape(NT, CHUNKS, CK)

    degp = _deg_kernel(srcp, dstp)[:, :, :, :1]
    xp = jnp.pad(x, ((0, NPAD - N), (0, 0)))

    t0 = _mm1(degp, xp, W_emb, b_emb.reshape(1, 512), W0)
    a0 = _agg4(t0, srcp, dstp)
    t1 = _mm_mid(degp, a0, b0.reshape(4, 1, 128), W1.reshape(4, 128, 512), 4)
    a1 = _agg4(t1, srcp, dstp)
    t2 = _mm_mid(degp, a1, b1.reshape(4, 1, 128), W2.reshape(4, 128, 128), 1)
    a2 = _agg1(t2, srcp, dstp)
    return _fin(degp, a2.reshape(2, NPAD, 128), b2.reshape(1, 128))


# speedup vs baseline: 6.1432x; 1.0289x over previous
import functools

import jax
import jax.numpy as jnp
from jax import lax
from jax.experimental import pallas as pl
from jax.experimental.pallas import tpu as pltpu
from jax.experimental.pallas import tpu_sc as plsc

N = 10000
NPAD = 10240
E = 160000
EPAD = 163840
NT = 32
CHUNKS = 80
CK = 64
STRIPE = NPAD // 16

_MESH = plsc.VectorSubcoreMesh(core_axis_name="c", subcore_axis_name="s")


@functools.partial(
    pl.kernel,
    out_type=jax.ShapeDtypeStruct((2, 2, NPAD, 128), jnp.float32),
    mesh=_MESH,
    scratch_types=[
        pltpu.VMEM((CHUNKS, CK), jnp.int32),
        pltpu.VMEM((CK, 128), jnp.float32),
        pltpu.VMEM((CK, 128), jnp.float32),
        pltpu.VMEM_SHARED((NPAD, 128), jnp.float32),
    ],
)
def _deg_kernel(src_hbm, dst_hbm, out_hbm, idx_v, ones_v, zeros_v, degs):
    c = lax.axis_index("c")
    s = lax.axis_index("s")
    wid = c * 16 + s

    def _fill(i, _):
        for k in range(128 // 16):
            ones_v[i, pl.ds(k * 16, 16)] = jnp.ones((16,), jnp.float32)
            zeros_v[i, pl.ds(k * 16, 16)] = jnp.zeros((16,), jnp.float32)
        return 0

    lax.fori_loop(0, CK, _fill, 0)

    base = s * STRIPE
    for phase, e_hbm in ((0, src_hbm), (1, dst_hbm)):
        pltpu.sync_copy(e_hbm.at[wid], idx_v)
        for z in range(STRIPE // CK):
            pltpu.sync_copy(zeros_v, degs.at[pl.ds(base + z * CK, CK)])
        plsc.subcore_barrier()

        def _scat(j, _):
            pltpu.sync_copy(ones_v, degs.at[idx_v.at[j]], add=True)
            return 0

        lax.fori_loop(0, CHUNKS, _scat, 0)
        plsc.subcore_barrier()
        pltpu.sync_copy(degs.at[pl.ds(base, STRIPE)],
                        out_hbm.at[c, phase, pl.ds(base, STRIPE)])
        plsc.subcore_barrier()


def _make_agg(nb):
    @functools.partial(
        pl.kernel,
        out_type=jax.ShapeDtypeStruct((2, nb, NPAD, 128), jnp.float32),
        mesh=_MESH,
        scratch_types=[
            pltpu.VMEM((CHUNKS, CK), jnp.int32),
            pltpu.VMEM((CHUNKS, CK), jnp.int32),
            pltpu.VMEM((CK, 128), jnp.float32),
            pltpu.VMEM((CK, 128), jnp.float32),
            pltpu.VMEM((CK, 128), jnp.float32),
            pltpu.VMEM_SHARED((NPAD, 128), jnp.float32),
            pltpu.SemaphoreType.DMA,
            pltpu.SemaphoreType.DMA,
            pltpu.SemaphoreType.DMA,
            pltpu.SemaphoreType.DMA,
        ],
    )
    def _agg_kernel(t_hbm, src_hbm, dst_hbm, out_hbm, sidx, didx,
                    rows_a, rows_b, zeros_v, aggs, sga, sgb, ssa, ssb):
        c = lax.axis_index("c")
        s = lax.axis_index("s")
        wid = c * 16 + s
        pltpu.sync_copy(src_hbm.at[wid], sidx)
        pltpu.sync_copy(dst_hbm.at[wid], didx)

        def _zero(i, _):
            for k in range(128 // 16):
                zeros_v[i, pl.ds(k * 16, 16)] = jnp.zeros((16,), jnp.float32)
            return 0

        lax.fori_loop(0, CK, _zero, 0)

        base = s * STRIPE
        for cb in range(nb):
            t_cb = t_hbm.at[cb]
            for z in range(STRIPE // CK):
                pltpu.sync_copy(zeros_v, aggs.at[pl.ds(base + z * CK, CK)])
            plsc.subcore_barrier()

            def _edge(g, _):
                j0 = g * 2
                j1 = j0 + 1
                ga = pltpu.async_copy(t_cb.at[sidx.at[j0]], rows_a, sga)
                gb = pltpu.async_copy(t_cb.at[sidx.at[j1]], rows_b, sgb)
                ga.wait()
                sa = pltpu.async_copy(rows_a, aggs.at[didx.at[j0]], ssa, add=True)
                gb.wait()
                sb = pltpu.async_copy(rows_b, aggs.at[didx.at[j1]], ssb, add=True)
                sa.wait()
                sb.wait()
                return 0

            lax.fori_loop(0, CHUNKS // 2, _edge, 0)
            plsc.subcore_barrier()
            pltpu.sync_copy(aggs.at[pl.ds(base, STRIPE)],
                            out_hbm.at[c, cb, pl.ds(base, STRIPE)])
            plsc.subcore_barrier()

    return _agg_kernel


_agg4 = _make_agg(4)
_agg1 = _make_agg(1)


def _norms(deg_ref, row0, rows):
    ridx = lax.broadcasted_iota(jnp.int32, (rows, 1), 0) + row0
    valid = ridx < N
    dego = deg_ref[0, 0] + deg_ref[1, 0]
    degi = deg_ref[0, 1] + deg_ref[1, 1]
    ns = jnp.where(valid, lax.rsqrt(jnp.maximum(dego, 1.0)), 0.0)
    nd = jnp.where(valid, lax.rsqrt(jnp.maximum(degi, 1.0)), 0.0)
    return ns, nd


_MMR = 2560


def _mm1_body(deg_ref, x_ref, we_ref, be_ref, w0_ref, out_ref):
    i = pl.program_id(0)
    ns, _ = _norms(deg_ref, i * _MMR, _MMR)
    h = jnp.dot(x_ref[...], we_ref[...], preferred_element_type=jnp.float32)
    h = (h + be_ref[...]) * ns
    w0 = w0_ref[...]
    for cb in range(4):
        out_ref[cb] = jnp.dot(h, w0[:, cb * 128:(cb + 1) * 128],
                              preferred_element_type=jnp.float32)


def _mm1(degp, xp, W_emb, b_emb, W0):
    return pl.pallas_call(
        _mm1_body,
        grid=(NPAD // _MMR,),
        in_specs=[
            pl.BlockSpec((2, 2, _MMR, 1), lambda i: (0, 0, i, 0)),
            pl.BlockSpec((_MMR, 256), lambda i: (i, 0)),
            pl.BlockSpec((256, 512), lambda i: (0, 0)),
            pl.BlockSpec((1, 512), lambda i: (0, 0)),
            pl.BlockSpec((512, 512), lambda i: (0, 0)),
        ],
        out_specs=pl.BlockSpec((4, _MMR, 128), lambda i: (0, i, 0)),
        out_shape=jax.ShapeDtypeStruct((4, NPAD, 128), jnp.float32),
    )(degp, xp, W_emb, b_emb, W0)


def _mm_mid_body(out_nb, deg_ref, a_ref, b_ref, w_ref, out_ref):
    i = pl.program_id(0)
    ns, nd = _norms(deg_ref, i * _MMR, _MMR)
    acc = jnp.zeros((_MMR, 512 if out_nb == 4 else 128), jnp.float32)
    for cb in range(4):
        u = (a_ref[0, cb] + a_ref[1, cb]) * nd + b_ref[cb]
        u = jnp.maximum(u, 0.0) * ns
        acc = acc + jnp.dot(u, w_ref[cb], preferred_element_type=jnp.float32)
    for cb in range(out_nb):
        out_ref[cb] = acc[:, cb * 128:(cb + 1) * 128]


def _mm_mid(degp, a, b, w, out_nb):
    return pl.pallas_call(
        functools.partial(_mm_mid_body, out_nb),
        grid=(NPAD // _MMR,),
        in_specs=[
            pl.BlockSpec((2, 2, _MMR, 1), lambda i: (0, 0, i, 0)),
            pl.BlockSpec((2, 4, _MMR, 128), lambda i: (0, 0, i, 0)),
            pl.BlockSpec((4, 1, 128), lambda i: (0, 0, 0)),
            pl.BlockSpec((4, 128, 128 * out_nb), lambda i: (0, 0, 0)),
        ],
        out_specs=pl.BlockSpec((out_nb, _MMR, 128), lambda i: (0, i, 0)),
        out_shape=jax.ShapeDtypeStruct((out_nb, NPAD, 128), jnp.float32),
    )(degp, a, b, w)


_FINR = 2000


def _fin_body(deg_ref, a_ref, b_ref, out_ref):
    i = pl.program_id(0)
    _, nd = _norms(deg_ref, i * _FINR, _FINR)
    out_ref[...] = (a_ref[0] + a_ref[1]) * nd + b_ref[...]


def _fin(degp, a2, b2):
    return pl.pallas_call(
        _fin_body,
        grid=(N // _FINR,),
        in_specs=[
            pl.BlockSpec((2, 2, _FINR, 1), lambda i: (0, 0, i, 0)),
            pl.BlockSpec((2, _FINR, 128), lambda i: (0, i, 0)),
            pl.BlockSpec((1, 128), lambda i: (0, 0)),
        ],
        out_specs=pl.BlockSpec((_FINR, 128), lambda i: (i, 0)),
        out_shape=jax.ShapeDtypeStruct((N, 128), jnp.float32),
    )(degp, a2, b2)


def kernel(x, edge_index, W_emb, b_emb, W0, b0, W1, b1, W2, b2):
    src = edge_index[0].astype(jnp.int32)
    dst = edge_index[1].astype(jnp.int32)
    pad = N + (jnp.arange(EPAD - E, dtype=jnp.int32) % 16)
    srcp = jnp.concatenate([src, pad]).reshape(NT, CHUNKS, CK)
    dstp = jnp.concatenate([dst, pad]).reshape(NT, CHUNKS, CK)

    degp = _deg_kernel(srcp, dstp)[:, :, :, :1]
    xp = jnp.pad(x, ((0, NPAD - N), (0, 0)))

    t0 = _mm1(degp, xp, W_emb, b_emb.reshape(1, 512), W0)
    a0 = _agg4(t0, srcp, dstp)
    t1 = _mm_mid(degp, a0, b0.reshape(4, 1, 128), W1.reshape(4, 128, 512), 4)
    a1 = _agg4(t1, srcp, dstp)
    t2 = _mm_mid(degp, a1, b1.reshape(4, 1, 128), W2.reshape(4, 128, 128), 1)
    a2 = _agg1(t2, srcp, dstp)
    return _fin(degp, a2.reshape(2, NPAD, 128), b2.reshape(1, 128))
